# Initial kernel scaffold; baseline (speedup 1.0000x reference)
import jax
import jax.numpy as jnp
from jax.experimental import pallas as pl


def kernel(x, edge_index, W1, b1, W2, b2):
    raise NotImplementedError("write your pallas kernel here")


# speedup vs baseline: 23.3297x; 23.3297x over previous
import jax
import jax.numpy as jnp
from jax import lax
from jax.experimental import pallas as pl
from jax.experimental.pallas import tpu as pltpu
from jax.experimental.pallas import tpu_sc as plsc

N = 10000
D = 128
E = 320000

NC = 2
NS = 16
NW = NC * NS

CHUNK = 128
CHUNKS_PER_TILE = 80
E_PAD = NW * CHUNKS_PER_TILE * CHUNK
N_PAD = 10112
ROWS_ZERO = N_PAD // NS
ROWS_OUT = ROWS_ZERO
DEG_W = 128

_MESH = plsc.VectorSubcoreMesh(core_axis_name="c", subcore_axis_name="s")


def _sc_degree_body(dst_hbm, ones_hbm, zeros_hbm, out_hbm, dst_v, ones_v, acc_sh):
    c = lax.axis_index("c")
    s = lax.axis_index("s")
    wid = c * NS + s
    pltpu.sync_copy(zeros_hbm, acc_sh.at[pl.ds(s * ROWS_ZERO, ROWS_ZERO)])
    pltpu.sync_copy(ones_hbm, ones_v)
    pltpu.sync_copy(dst_hbm.at[pl.ds(wid * CHUNKS_PER_TILE, CHUNKS_PER_TILE)], dst_v)
    plsc.subcore_barrier()

    def body(j, carry):
        pltpu.sync_copy(ones_v, acc_sh.at[dst_v.at[j]], add=True)
        return carry

    lax.fori_loop(0, CHUNKS_PER_TILE, body, 0)
    plsc.subcore_barrier()
    pltpu.sync_copy(acc_sh.at[pl.ds(s * ROWS_OUT, ROWS_OUT)],
                    out_hbm.at[c, pl.ds(s * ROWS_OUT, ROWS_OUT)])


NBUF = 2
NSPLIT = 2
SUB = CHUNK // NSPLIT
HALF = CHUNKS_PER_TILE // 2
RING_STEPS = HALF // NBUF


def _issue_gathers(z_hbm, src_v, bufs_v, gsem, j, b):
    for h in range(NSPLIT):
        pltpu.async_copy(z_hbm.at[src_v.at[j, pl.ds(h * SUB, SUB)]],
                         bufs_v.at[b, pl.ds(h * SUB, SUB)], gsem.at[b, h])


def _wait_gathers(z_hbm, src_v, bufs_v, gsem, j, b):
    for h in range(NSPLIT):
        pltpu.make_async_copy(z_hbm.at[src_v.at[j, pl.ds(h * SUB, SUB)]],
                              bufs_v.at[b, pl.ds(h * SUB, SUB)],
                              gsem.at[b, h]).wait()


def _ring_phase(z_hbm, acc_sh, src_v, dst_v, bufs_v, gsem, ssem):
    for b in range(NBUF):
        _issue_gathers(z_hbm, src_v, bufs_v, gsem, b, b)

    def body(j0, carry):
        for b in range(NBUF):
            j = j0 * NBUF + b
            _wait_gathers(z_hbm, src_v, bufs_v, gsem, j, b)
            pltpu.async_copy(bufs_v.at[b], acc_sh.at[dst_v.at[j]], ssem.at[b],
                             add=True)
            pltpu.make_async_copy(bufs_v.at[b], acc_sh.at[dst_v.at[j]],
                                  ssem.at[b]).wait()
            _issue_gathers(z_hbm, src_v, bufs_v, gsem, j + NBUF, b)
        return carry

    lax.fori_loop(0, RING_STEPS - 1, body, 0)
    for b in range(NBUF):
        j = (RING_STEPS - 1) * NBUF + b
        _wait_gathers(z_hbm, src_v, bufs_v, gsem, j, b)
        pltpu.async_copy(bufs_v.at[b], acc_sh.at[dst_v.at[j]], ssem.at[b],
                         add=True)
    for b in range(NBUF):
        j = (RING_STEPS - 1) * NBUF + b
        pltpu.make_async_copy(bufs_v.at[b], acc_sh.at[dst_v.at[j]],
                              ssem.at[b]).wait()


def _sc_scatter_body(z_hbm, src_hbm, dst_hbm, zeros_hbm, out_hbm,
                     src_v, dst_v, bufs_v, acc_sh, gsem, ssem):
    c = lax.axis_index("c")
    s = lax.axis_index("s")
    wid = c * NS + s
    pltpu.sync_copy(zeros_hbm, acc_sh.at[pl.ds(s * ROWS_ZERO, ROWS_ZERO)])
    base = wid * CHUNKS_PER_TILE
    pltpu.sync_copy(src_hbm.at[pl.ds(base, HALF)], src_v)
    pltpu.sync_copy(dst_hbm.at[pl.ds(base, HALF)], dst_v)
    plsc.subcore_barrier()
    _ring_phase(z_hbm, acc_sh, src_v, dst_v, bufs_v, gsem, ssem)
    pltpu.sync_copy(src_hbm.at[pl.ds(base + HALF, HALF)], src_v)
    pltpu.sync_copy(dst_hbm.at[pl.ds(base + HALF, HALF)], dst_v)
    _ring_phase(z_hbm, acc_sh, src_v, dst_v, bufs_v, gsem, ssem)
    plsc.subcore_barrier()
    pltpu.sync_copy(acc_sh.at[pl.ds(s * ROWS_OUT, ROWS_OUT)],
                    out_hbm.at[c, pl.ds(s * ROWS_OUT, ROWS_OUT)])


_sc_degree = pl.kernel(
    _sc_degree_body,
    out_type=jax.ShapeDtypeStruct((NC, N_PAD, DEG_W), jnp.float32),
    mesh=_MESH,
    scratch_types=[
        pltpu.VMEM((CHUNKS_PER_TILE, CHUNK), jnp.int32),
        pltpu.VMEM((CHUNK, DEG_W), jnp.float32),
        pltpu.VMEM_SHARED((N_PAD, DEG_W), jnp.float32),
    ],
)

_sc_scatter = pl.kernel(
    _sc_scatter_body,
    out_type=jax.ShapeDtypeStruct((NC, N_PAD, D), jnp.float32),
    mesh=_MESH,
    scratch_types=[
        pltpu.VMEM((HALF, CHUNK), jnp.int32),
        pltpu.VMEM((HALF, CHUNK), jnp.int32),
        pltpu.VMEM((NBUF, CHUNK, D), jnp.float32),
        pltpu.VMEM_SHARED((N_PAD, D), jnp.float32),
        pltpu.SemaphoreType.DMA((NBUF, NSPLIT)),
        pltpu.SemaphoreType.DMA((NBUF,)),
    ],
)


BM = 2000
GRID = N // BM


def _tc1_body(deg_ref, x_ref, w_ref, z_ref, dis_ref):
    deg = deg_ref[:, 0:1] + deg_ref[:, 1:2] + 1.0
    dis = lax.rsqrt(deg)
    xw = jnp.dot(x_ref[...], w_ref[...], preferred_element_type=jnp.float32)
    z_ref[...] = dis * xw
    dis_ref[...] = dis


def _tc2_body(p_ref, z_ref, dis_ref, b_ref, w_ref, z2_ref):
    agg = p_ref[0] + p_ref[1] + z_ref[...]
    h = jnp.maximum(dis_ref[...] * agg + b_ref[...], 0.0)
    z2_ref[...] = dis_ref[...] * jnp.dot(h, w_ref[...],
                                         preferred_element_type=jnp.float32)


def _tc3_body(p_ref, z_ref, dis_ref, b_ref, out_ref):
    agg = p_ref[0] + p_ref[1] + z_ref[...]
    out_ref[...] = jnp.maximum(dis_ref[...] * agg + b_ref[...], 0.0)


_tc1 = pl.pallas_call(
    _tc1_body,
    grid=(GRID,),
    in_specs=[
        pl.BlockSpec((BM, NC), lambda i: (i, 0)),
        pl.BlockSpec((BM, D), lambda i: (i, 0)),
        pl.BlockSpec((D, D), lambda i: (0, 0)),
    ],
    out_specs=[
        pl.BlockSpec((BM, D), lambda i: (i, 0)),
        pl.BlockSpec((BM, 1), lambda i: (i, 0)),
    ],
    out_shape=[
        jax.ShapeDtypeStruct((N, D), jnp.float32),
        jax.ShapeDtypeStruct((N, 1), jnp.float32),
    ],
)

_tc2 = pl.pallas_call(
    _tc2_body,
    grid=(GRID,),
    in_specs=[
        pl.BlockSpec((NC, BM, D), lambda i: (0, i, 0)),
        pl.BlockSpec((BM, D), lambda i: (i, 0)),
        pl.BlockSpec((BM, 1), lambda i: (i, 0)),
        pl.BlockSpec((1, D), lambda i: (0, 0)),
        pl.BlockSpec((D, D), lambda i: (0, 0)),
    ],
    out_specs=pl.BlockSpec((BM, D), lambda i: (i, 0)),
    out_shape=jax.ShapeDtypeStruct((N, D), jnp.float32),
)

_tc3 = pl.pallas_call(
    _tc3_body,
    grid=(GRID,),
    in_specs=[
        pl.BlockSpec((NC, BM, D), lambda i: (0, i, 0)),
        pl.BlockSpec((BM, D), lambda i: (i, 0)),
        pl.BlockSpec((BM, 1), lambda i: (i, 0)),
        pl.BlockSpec((1, D), lambda i: (0, 0)),
    ],
    out_specs=pl.BlockSpec((BM, D), lambda i: (i, 0)),
    out_shape=jax.ShapeDtypeStruct((N, D), jnp.float32),
)


def kernel(x, edge_index, W1, b1, W2, b2):
    src = edge_index[0].astype(jnp.int32)
    dst = edge_index[1].astype(jnp.int32)
    pad = E_PAD - E
    pad_src = jnp.arange(pad, dtype=jnp.int32) % N
    pad_dst = N + jnp.arange(pad, dtype=jnp.int32) % (N_PAD - N)
    src2d = jnp.concatenate([src, pad_src]).reshape(E_PAD // CHUNK, CHUNK)
    dst2d = jnp.concatenate([dst, pad_dst]).reshape(E_PAD // CHUNK, CHUNK)

    ones_deg = jnp.ones((CHUNK, DEG_W), jnp.float32)
    zeros_deg = jnp.zeros((ROWS_ZERO, DEG_W), jnp.float32)
    zeros_row = jnp.zeros((ROWS_ZERO, D), jnp.float32)

    deg_p = _sc_degree(dst2d, ones_deg, zeros_deg)
    deg_t = deg_p[:, :N, 0].T

    z1, dis = _tc1(deg_t, x, W1)
    p1 = _sc_scatter(z1, src2d, dst2d, zeros_row)
    z2 = _tc2(p1, z1, dis, b1.reshape(1, D), W2)
    p2 = _sc_scatter(z2, src2d, dst2d, zeros_row)
    out = _tc3(p2, z2, dis, b2.reshape(1, D))
    return out


# speedup vs baseline: 27.5406x; 1.1805x over previous
import jax
import jax.numpy as jnp
from jax import lax
from jax.experimental import pallas as pl
from jax.experimental.pallas import tpu as pltpu
from jax.experimental.pallas import tpu_sc as plsc

N = 10000
D = 128
E = 320000

NC = 2
NS = 16
NW = NC * NS

CHUNK = 128
CHUNKS_PER_TILE = 80
E_PAD = NW * CHUNKS_PER_TILE * CHUNK
N_PAD = 10112
ROWS_ZERO = N_PAD // NS
ROWS_OUT = ROWS_ZERO
DEG_W = 128

_MESH = plsc.VectorSubcoreMesh(core_axis_name="c", subcore_axis_name="s")


DEG_K = 8


def _sc_degree_body(dst_hbm, ones_hbm, zeros_hbm, out_hbm, dst_v, ones_v,
                    acc_sh, dsem):
    c = lax.axis_index("c")
    s = lax.axis_index("s")
    wid = c * NS + s
    pltpu.sync_copy(zeros_hbm, acc_sh.at[pl.ds(s * ROWS_ZERO, ROWS_ZERO)])
    pltpu.sync_copy(ones_hbm, ones_v)
    pltpu.sync_copy(dst_hbm.at[pl.ds(wid * CHUNKS_PER_TILE, CHUNKS_PER_TILE)], dst_v)
    plsc.subcore_barrier()

    for b in range(DEG_K):
        pltpu.async_copy(ones_v, acc_sh.at[dst_v.at[b]], dsem.at[b], add=True)

    def body(j0, carry):
        for b in range(DEG_K):
            j = j0 * DEG_K + b
            pltpu.make_async_copy(ones_v, acc_sh.at[dst_v.at[j]],
                                  dsem.at[b]).wait()
            pltpu.async_copy(ones_v, acc_sh.at[dst_v.at[j + DEG_K]],
                             dsem.at[b], add=True)
        return carry

    lax.fori_loop(0, CHUNKS_PER_TILE // DEG_K - 1, body, 0)
    for b in range(DEG_K):
        j = CHUNKS_PER_TILE - DEG_K + b
        pltpu.make_async_copy(ones_v, acc_sh.at[dst_v.at[j]], dsem.at[b]).wait()
    plsc.subcore_barrier()
    pltpu.sync_copy(acc_sh.at[pl.ds(s * ROWS_OUT, ROWS_OUT)],
                    out_hbm.at[c, pl.ds(s * ROWS_OUT, ROWS_OUT)])


NBUF = 2
NSPLIT = 2
SUB = CHUNK // NSPLIT
HALF = CHUNKS_PER_TILE // 2
RING_STEPS = HALF // NBUF


def _issue_gathers(z_hbm, src_v, bufs_v, gsem, j, b):
    for h in range(NSPLIT):
        pltpu.async_copy(z_hbm.at[src_v.at[j, pl.ds(h * SUB, SUB)]],
                         bufs_v.at[b, pl.ds(h * SUB, SUB)], gsem.at[b, h])


def _wait_gathers(z_hbm, src_v, bufs_v, gsem, j, b):
    for h in range(NSPLIT):
        pltpu.make_async_copy(z_hbm.at[src_v.at[j, pl.ds(h * SUB, SUB)]],
                              bufs_v.at[b, pl.ds(h * SUB, SUB)],
                              gsem.at[b, h]).wait()


def _ring_phase(z_hbm, acc_sh, src_v, dst_v, bufs_v, gsem, ssem):
    for b in range(NBUF):
        _issue_gathers(z_hbm, src_v, bufs_v, gsem, b, b)

    def body(j0, carry):
        for b in range(NBUF):
            j = j0 * NBUF + b
            _wait_gathers(z_hbm, src_v, bufs_v, gsem, j, b)
            pltpu.async_copy(bufs_v.at[b], acc_sh.at[dst_v.at[j]], ssem.at[b],
                             add=True)
            pltpu.make_async_copy(bufs_v.at[b], acc_sh.at[dst_v.at[j]],
                                  ssem.at[b]).wait()
            _issue_gathers(z_hbm, src_v, bufs_v, gsem, j + NBUF, b)
        return carry

    lax.fori_loop(0, RING_STEPS - 1, body, 0)
    for b in range(NBUF):
        j = (RING_STEPS - 1) * NBUF + b
        _wait_gathers(z_hbm, src_v, bufs_v, gsem, j, b)
        pltpu.async_copy(bufs_v.at[b], acc_sh.at[dst_v.at[j]], ssem.at[b],
                         add=True)
    for b in range(NBUF):
        j = (RING_STEPS - 1) * NBUF + b
        pltpu.make_async_copy(bufs_v.at[b], acc_sh.at[dst_v.at[j]],
                              ssem.at[b]).wait()


def _sc_scatter_body(z_hbm, src_hbm, dst_hbm, zeros_hbm, out_hbm,
                     src_v, dst_v, bufs_v, acc_sh, gsem, ssem):
    c = lax.axis_index("c")
    s = lax.axis_index("s")
    wid = c * NS + s
    pltpu.sync_copy(zeros_hbm, acc_sh.at[pl.ds(s * ROWS_ZERO, ROWS_ZERO)])
    base = wid * CHUNKS_PER_TILE
    pltpu.sync_copy(src_hbm.at[pl.ds(base, HALF)], src_v)
    pltpu.sync_copy(dst_hbm.at[pl.ds(base, HALF)], dst_v)
    plsc.subcore_barrier()
    _ring_phase(z_hbm, acc_sh, src_v, dst_v, bufs_v, gsem, ssem)
    pltpu.sync_copy(src_hbm.at[pl.ds(base + HALF, HALF)], src_v)
    pltpu.sync_copy(dst_hbm.at[pl.ds(base + HALF, HALF)], dst_v)
    _ring_phase(z_hbm, acc_sh, src_v, dst_v, bufs_v, gsem, ssem)
    plsc.subcore_barrier()
    pltpu.sync_copy(acc_sh.at[pl.ds(s * ROWS_OUT, ROWS_OUT)],
                    out_hbm.at[c, pl.ds(s * ROWS_OUT, ROWS_OUT)])


_sc_degree = pl.kernel(
    _sc_degree_body,
    out_type=jax.ShapeDtypeStruct((NC, N_PAD, DEG_W), jnp.float32),
    mesh=_MESH,
    scratch_types=[
        pltpu.VMEM((CHUNKS_PER_TILE, CHUNK), jnp.int32),
        pltpu.VMEM((CHUNK, DEG_W), jnp.float32),
        pltpu.VMEM_SHARED((N_PAD, DEG_W), jnp.float32),
        pltpu.SemaphoreType.DMA((DEG_K,)),
    ],
)

_sc_scatter = pl.kernel(
    _sc_scatter_body,
    out_type=jax.ShapeDtypeStruct((NC, N_PAD, D), jnp.float32),
    mesh=_MESH,
    scratch_types=[
        pltpu.VMEM((HALF, CHUNK), jnp.int32),
        pltpu.VMEM((HALF, CHUNK), jnp.int32),
        pltpu.VMEM((NBUF, CHUNK, D), jnp.float32),
        pltpu.VMEM_SHARED((N_PAD, D), jnp.float32),
        pltpu.SemaphoreType.DMA((NBUF, NSPLIT)),
        pltpu.SemaphoreType.DMA((NBUF,)),
    ],
)


BM = 2000
GRID = N // BM


def _tc1_body(deg_ref, x_ref, w_ref, z_ref, dis_ref):
    deg = deg_ref[0, :, 0:1] + deg_ref[1, :, 0:1] + 1.0
    dis = lax.rsqrt(deg)
    xw = jnp.dot(x_ref[...], w_ref[...], preferred_element_type=jnp.float32)
    z_ref[...] = dis * xw
    dis_ref[...] = dis


def _tc2_body(p_ref, z_ref, dis_ref, b_ref, w_ref, z2_ref):
    agg = p_ref[0] + p_ref[1] + z_ref[...]
    h = jnp.maximum(dis_ref[...] * agg + b_ref[...], 0.0)
    z2_ref[...] = dis_ref[...] * jnp.dot(h, w_ref[...],
                                         preferred_element_type=jnp.float32)


def _tc3_body(p_ref, z_ref, dis_ref, b_ref, out_ref):
    agg = p_ref[0] + p_ref[1] + z_ref[...]
    out_ref[...] = jnp.maximum(dis_ref[...] * agg + b_ref[...], 0.0)


_tc1 = pl.pallas_call(
    _tc1_body,
    grid=(GRID,),
    in_specs=[
        pl.BlockSpec((NC, BM, DEG_W), lambda i: (0, i, 0)),
        pl.BlockSpec((BM, D), lambda i: (i, 0)),
        pl.BlockSpec((D, D), lambda i: (0, 0)),
    ],
    out_specs=[
        pl.BlockSpec((BM, D), lambda i: (i, 0)),
        pl.BlockSpec((BM, 1), lambda i: (i, 0)),
    ],
    out_shape=[
        jax.ShapeDtypeStruct((N, D), jnp.float32),
        jax.ShapeDtypeStruct((N, 1), jnp.float32),
    ],
)

_tc2 = pl.pallas_call(
    _tc2_body,
    grid=(GRID,),
    in_specs=[
        pl.BlockSpec((NC, BM, D), lambda i: (0, i, 0)),
        pl.BlockSpec((BM, D), lambda i: (i, 0)),
        pl.BlockSpec((BM, 1), lambda i: (i, 0)),
        pl.BlockSpec((1, D), lambda i: (0, 0)),
        pl.BlockSpec((D, D), lambda i: (0, 0)),
    ],
    out_specs=pl.BlockSpec((BM, D), lambda i: (i, 0)),
    out_shape=jax.ShapeDtypeStruct((N, D), jnp.float32),
)

_tc3 = pl.pallas_call(
    _tc3_body,
    grid=(GRID,),
    in_specs=[
        pl.BlockSpec((NC, BM, D), lambda i: (0, i, 0)),
        pl.BlockSpec((BM, D), lambda i: (i, 0)),
        pl.BlockSpec((BM, 1), lambda i: (i, 0)),
        pl.BlockSpec((1, D), lambda i: (0, 0)),
    ],
    out_specs=pl.BlockSpec((BM, D), lambda i: (i, 0)),
    out_shape=jax.ShapeDtypeStruct((N, D), jnp.float32),
)


def kernel(x, edge_index, W1, b1, W2, b2):
    src = edge_index[0].astype(jnp.int32)
    dst = edge_index[1].astype(jnp.int32)
    pad = E_PAD - E
    pad_src = jnp.arange(pad, dtype=jnp.int32) % N
    pad_dst = N + jnp.arange(pad, dtype=jnp.int32) % (N_PAD - N)
    src2d = jnp.concatenate([src, pad_src]).reshape(E_PAD // CHUNK, CHUNK)
    dst2d = jnp.concatenate([dst, pad_dst]).reshape(E_PAD // CHUNK, CHUNK)

    ones_deg = jnp.ones((CHUNK, DEG_W), jnp.float32)
    zeros_deg = jnp.zeros((ROWS_ZERO, DEG_W), jnp.float32)
    zeros_row = jnp.zeros((ROWS_ZERO, D), jnp.float32)

    deg_p = _sc_degree(dst2d, ones_deg, zeros_deg)

    z1, dis = _tc1(deg_p, x, W1)
    p1 = _sc_scatter(z1, src2d, dst2d, zeros_row)
    z2 = _tc2(p1, z1, dis, b1.reshape(1, D), W2)
    p2 = _sc_scatter(z2, src2d, dst2d, zeros_row)
    out = _tc3(p2, z2, dis, b2.reshape(1, D))
    return out
